# baseline (speedup 1.0000x reference)
import functools

import jax
import jax.numpy as jnp
from jax import lax
from jax.experimental import pallas as pl
from jax.experimental.pallas import tpu as pltpu, tpu_sc as plsc

N = 8
C = 19
HW = 512 * 512
POS = 0.9
NEG = 0.005

NC = 2
NS = 16
L = 16
NW = NC * NS
Q = (N * HW) // NW
S = 2048
CHUNKS = Q // S


def _body(label_hbm, out_hbm, labv, outv):
    cid = lax.axis_index("c")
    sid = lax.axis_index("s")
    wid = sid * NC + cid
    n = wid // (NW // N)
    q = wid % (NW // N)
    row0 = n * C
    col_base = q * Q

    neg16 = jnp.full((L,), NEG, jnp.float32)
    pos16 = jnp.full((L,), POS, jnp.float32)
    lane = lax.iota(jnp.int32, L)

    def fill(k, carry):
        outv[pl.ds(k * L, L)] = neg16
        return carry

    lax.fori_loop(0, (C * S) // L, fill, None)

    def scatter_val(val16):
        def go(j, carry):
            lab = labv[pl.ds(j * L, L)]
            idx = jnp.minimum(lab, C - 1) * S + (j * L + lane)
            plsc.store_scatter(outv, [idx], val16)
            return carry
        lax.fori_loop(0, S // L, go, None)

    def chunk(i, carry):
        col0 = col_base + i * S
        pltpu.sync_copy(label_hbm.at[pl.ds(n * HW + col0, S)], labv)
        scatter_val(pos16)
        for c in range(C):
            pltpu.sync_copy(outv.at[pl.ds(c * S, S)],
                            out_hbm.at[row0 + c, pl.ds(col0, S)])
        scatter_val(neg16)
        return carry

    lax.fori_loop(0, CHUNKS, chunk, None)


@jax.jit
def kernel(label):
    lab_flat = label.reshape(-1)
    run = pl.kernel(
        _body,
        out_type=jax.ShapeDtypeStruct((N * C, HW), jnp.float32),
        mesh=plsc.VectorSubcoreMesh(core_axis_name="c", subcore_axis_name="s",
                                    num_cores=NC, num_subcores=NS),
        scratch_types=[
            pltpu.VMEM((S,), jnp.int32),
            pltpu.VMEM((C * S,), jnp.float32),
        ],
        compiler_params=pltpu.CompilerParams(needs_layout_passes=False),
    )
    out = run(lab_flat)
    return out.reshape(N, C, 512, 512)
